# baseline (speedup 1.0000x reference)
import functools

import jax
import jax.numpy as jnp
from jax import lax
from jax.experimental import pallas as pl
from jax.experimental.pallas import tpu as pltpu
from jax.experimental.pallas import tpu_sc as plsc


@functools.lru_cache(maxsize=None)
def _build_gather(B: int, T: int, ROW: int):
    info = plsc.get_sparse_core_info()
    NW = info.num_cores * info.num_subcores
    b_per_w = B // NW
    CH = min(16, b_per_w)
    n_ch = b_per_w // CH
    mesh = plsc.VectorSubcoreMesh(core_axis_name="c", subcore_axis_name="s")

    @functools.partial(
        pl.kernel,
        mesh=mesh,
        out_type=jax.ShapeDtypeStruct((B, ROW), jnp.float32),
        scratch_types=[
            pltpu.VMEM((2, CH), jnp.int32),
            pltpu.VMEM((2, CH, ROW), jnp.float32),
            pltpu.SemaphoreType.DMA,
            pltpu.SemaphoreType.DMA,
        ],
    )
    def gather_kernel(table_hbm, idx_hbm, out_hbm, idx_v, rows_v, gsem, ssem):
        wid = lax.axis_index("s") * info.num_cores + lax.axis_index("c")
        base = wid * b_per_w

        def start_gather(c, slot):
            pltpu.sync_copy(idx_hbm.at[pl.ds(base + c * CH, CH)], idx_v.at[slot])
            return pltpu.async_copy(table_hbm.at[idx_v.at[slot]], rows_v.at[slot], gsem)

        g = start_gather(0, 0)
        s_prev = None
        for c in range(n_ch):
            slot = c % 2
            g.wait()
            if s_prev is not None:
                s_prev.wait()
            if c + 1 < n_ch:
                g = start_gather(c + 1, 1 - slot)
            s_prev = pltpu.async_copy(
                rows_v.at[slot], out_hbm.at[pl.ds(base + c * CH, CH)], ssem
            )
        s_prev.wait()

    return gather_kernel


def kernel(query_tokens, task_ids, batch_size):
    T, Tq, D = query_tokens.shape
    B = task_ids.shape[0]
    table = query_tokens.reshape(T, Tq * D)
    idx = task_ids.astype(jnp.int32)
    out = _build_gather(B, T, Tq * D)(table, idx)
    return out.reshape(B, Tq, D)
